# baseline (speedup 1.0000x reference)
import functools

import jax
import jax.numpy as jnp
from jax import lax
from jax.experimental import pallas as pl
from jax.experimental.pallas import tpu as pltpu
from jax.experimental.pallas import tpu_sc as plsc

D = 128
NW = 32
LANES = 128
G = 1
NBUF = 5


def _i32(x):
    return jnp.int32(x)


@functools.lru_cache(maxsize=None)
def _make_gather(n_rows: int):
    assert n_rows % (NW * G * LANES) == 0
    C = n_rows // (NW * G * LANES)
    assert C > 2 * NBUF
    K = (C - NBUF) // NBUF
    mesh = plsc.VectorSubcoreMesh(core_axis_name="c", subcore_axis_name="s")

    @functools.partial(
        pl.kernel,
        mesh=mesh,
        out_type=jax.ShapeDtypeStruct((NW * C, G * LANES, D), jnp.float32),
        scratch_types=[
            pltpu.VMEM((C, G * LANES), jnp.int32),
            pltpu.VMEM((NBUF, G * LANES, D), jnp.float32),
        ]
        + [pltpu.SemaphoreType.DMA] * (2 * NBUF),
    )
    def gather_kernel(idx_hbm, table_hbm, out_hbm, idx_v, bufs, *sems):
        gsem = sems[:NBUF]
        ssem = sems[NBUF:]
        wid = lax.axis_index("s") * 2 + lax.axis_index("c")
        cbase = wid * _i32(C)
        pltpu.sync_copy(idx_hbm.at[wid], idx_v)

        def start_gather(chunk, b):
            pltpu.async_copy(
                table_hbm.at[idx_v.at[chunk]], bufs.at[_i32(b)], gsem[b])

        def start_store(chunk, b):
            pltpu.async_copy(
                bufs.at[_i32(b)], out_hbm.at[cbase + chunk], ssem[b])

        def wait_gather(b):
            pltpu.make_async_copy(
                table_hbm.at[idx_v.at[_i32(0)]], bufs.at[_i32(b)], gsem[b]
            ).wait()

        def wait_store(b):
            pltpu.make_async_copy(
                bufs.at[_i32(b)], out_hbm.at[cbase], ssem[b]).wait()

        def recycle(j, b):
            wait_store(b)
            start_gather(j + _i32(NBUF - 1), b)

        def retire(j, b):
            wait_gather(b)
            start_store(j, b)

        for b in range(NBUF):
            start_gather(_i32(b), b)
        retire(_i32(0), 0)

        @pl.loop(_i32(1), _i32(1 + K * NBUF), step=_i32(NBUF))
        def body(jj):
            for u in range(NBUF):
                j = jj + _i32(u)
                recycle(j, u % NBUF)
                retire(j, (1 + u) % NBUF)

        for j in range(1 + K * NBUF, C - NBUF + 1):
            recycle(_i32(j), (j - 1) % NBUF)
            retire(_i32(j), j % NBUF)

        for j in range(C - NBUF + 1, C):
            retire(_i32(j), j % NBUF)
        for b in range(NBUF):
            wait_store(b)

    return gather_kernel


def kernel(input_seq, table):
    B, S = input_seq.shape
    n = B * S
    idx = input_seq.astype(jnp.int32).reshape(
        NW, n // (NW * G * LANES), G * LANES)
    table = table.astype(jnp.float32)
    out = _make_gather(n)(idx, table)
    return out.reshape(B, S, D)
